# baseline (speedup 1.0000x reference)
import functools

import jax
import jax.numpy as jnp
from jax import lax
from jax.experimental import pallas as pl
from jax.experimental.pallas import tpu as pltpu
from jax.experimental.pallas import tpu_sc as plsc

N = 10000
E = 320000
D_IN = 128
D_HID = 64
D_OUT = 128

NC = 2
NS = 16
NW = NC * NS
EPW = E // NW

GC = 200
GSUB = 40
NSUB = GC // GSUB
NCHG = EPW // GC

BE = 2000

R = 320
NP = NW * R
CD = 8000
NCH = E // CD
SCU = 5
DR = 64
LCAP = 8096
LSH = 9


def _node_mlp_body(x_ref, w1_ref, b1_ref, p_ref):
  x = x_ref[...]
  w1 = w1_ref[...]
  wa = w1[:D_IN, :]
  wb = w1[D_IN:, :]
  wcat = jnp.concatenate([wa - wb, wb], axis=1)
  bcat = jnp.concatenate(
      [b1_ref[...], jnp.zeros((1, D_HID), jnp.float32)], axis=1
  )
  p_ref[...] = jnp.dot(x, wcat, preferred_element_type=jnp.float32) + bcat


def _node_mlp(x, W1, b1):
  return pl.pallas_call(
      _node_mlp_body,
      out_shape=jax.ShapeDtypeStruct((N, 2 * D_HID), jnp.float32),
  )(x, W1, b1.reshape(1, D_HID))


def _sc_gather_body(dst_hbm, src_hbm, p_hbm, r_hbm,
                    dia, sia, pd0, ps0, pd1, ps1, gs0, gs1, ws0, ws1):
  w = lax.axis_index("s") * NC + lax.axis_index("c")
  base = w * EPW
  pltpu.sync_copy(dst_hbm.at[pl.ds(base, EPW)], dia)
  pltpu.sync_copy(src_hbm.at[pl.ds(base, EPW)], sia)

  pdb = (pd0, pd1)
  psb = (ps0, ps1)
  gsb = (gs0, gs1)
  wsb = (ws0, ws1)

  def fire(ci, b):
    for k in range(NSUB):
      isl = pl.ds(ci * GC + k * GSUB, GSUB)
      sl = pl.ds(k * GSUB, GSUB)
      pltpu.async_copy(p_hbm.at[dia.at[isl]], pdb[b].at[sl], gsb[b])
      pltpu.async_copy(p_hbm.at[sia.at[isl]], psb[b].at[sl], gsb[b])

  def drain_gather(b):
    pltpu.make_async_copy(p_hbm.at[pl.ds(0, GC)], pdb[b], gsb[b]).wait()
    pltpu.make_async_copy(p_hbm.at[pl.ds(0, GC)], psb[b], gsb[b]).wait()

  def drain_wb(b):
    pltpu.make_async_copy(p_hbm.at[pl.ds(0, GC)], pdb[b], wsb[b]).wait()

  fire(0, 0)

  def step(ci2, _):
    for b in range(2):
      cur = ci2 * 2 + b

      @pl.when(cur >= 1)
      def _():
        drain_wb(1 - b)

      @pl.when(cur + 1 < NCHG)
      def _():
        fire(cur + 1, 1 - b)

      drain_gather(b)

      def relu4(i4, _):
        for r2 in range(4):
          i = i4 * 4 + r2
          for q in range(D_HID // 16):
            sl = pl.ds(q * 16, 16)
            v = pdb[b][i, sl] + psb[b][i, pl.ds(D_HID + q * 16, 16)]
            pdb[b][i, sl] = jnp.maximum(v, 0.0)
        return 0

      lax.fori_loop(0, GC // 4, relu4, 0)
      pltpu.async_copy(pdb[b], r_hbm.at[pl.ds(base + cur * GC, GC)], wsb[b])
    return 0

  lax.fori_loop(0, NCHG // 2, step, 0)
  drain_wb((NCHG - 1) % 2)


def _sc_gather(dst, src, P):
  mesh = plsc.VectorSubcoreMesh(core_axis_name="c", subcore_axis_name="s")
  f = functools.partial(
      pl.kernel,
      out_type=jax.ShapeDtypeStruct((E, 2 * D_HID), jnp.float32),
      mesh=mesh,
      compiler_params=pltpu.CompilerParams(needs_layout_passes=False),
      scratch_types=[
          pltpu.VMEM((EPW,), jnp.int32),
          pltpu.VMEM((EPW,), jnp.int32),
          pltpu.VMEM((GC, 2 * D_HID), jnp.float32),
          pltpu.VMEM((GC, 2 * D_HID), jnp.float32),
          pltpu.VMEM((GC, 2 * D_HID), jnp.float32),
          pltpu.VMEM((GC, 2 * D_HID), jnp.float32),
          pltpu.SemaphoreType.DMA,
          pltpu.SemaphoreType.DMA,
          pltpu.SemaphoreType.DMA,
          pltpu.SemaphoreType.DMA,
      ],
  )(_sc_gather_body)
  return f(dst, src, P)


def _mlp2_body(r_ref, w2_ref, h_ref):
  h_ref[...] = jnp.dot(
      r_ref[:, :D_HID], w2_ref[...], preferred_element_type=jnp.float32
  )


def _mlp2(r, W2):
  return pl.pallas_call(
      _mlp2_body,
      grid=(E // BE,),
      in_specs=[
          pl.BlockSpec((BE, 2 * D_HID), lambda i: (i, 0)),
          pl.BlockSpec((D_HID, D_OUT), lambda i: (0, 0)),
      ],
      out_specs=pl.BlockSpec((BE, D_OUT), lambda i: (i, 0)),
      out_shape=jax.ShapeDtypeStruct((E, D_OUT), jnp.float32),
  )(r, W2)


def _sc_scatter_body(dst_hbm, h_hbm, b2_hbm, out_hbm,
                     tbl, db0, db1, plist, gb0, gb1, eb0, eb1, b2v,
                     ds0, ds1, gs0, gs1):
  w = lax.axis_index("s") * NC + lax.axis_index("c")
  lo = w * R

  def init_row(i, _):
    for q in range(D_OUT // 16):
      tbl[i, pl.ds(q * 16, 16)] = jnp.full((16,), -jnp.inf, jnp.float32)
    return 0

  lax.fori_loop(0, R, init_row, 0)

  for v in range(DR // 16):
    plist[pl.ds(v * 16, 16)] = jnp.zeros((16,), jnp.int32)

  dbb = (db0, db1)
  dsb = (ds0, ds1)
  gbb = (gb0, gb1)
  gsb = (gs0, gs1)
  iota = lax.iota(jnp.int32, 16)

  def fire_dst(ci, b):
    pltpu.async_copy(dst_hbm.at[pl.ds(ci * CD, CD)], dbb[b], dsb[b])

  def drain_dst(b):
    pltpu.make_async_copy(dst_hbm.at[pl.ds(0, CD)], dbb[b], dsb[b]).wait()

  ebb = (eb0, eb1)

  def fire_gather(list_base, b):
    for v in range(DR // 16):
      pv = plist[pl.ds(list_base + v * 16, 16)]
      ebb[b][pl.ds(v * 16, 16)] = lax.shift_right_logical(pv, LSH)
    pltpu.async_copy(h_hbm.at[ebb[b]], gbb[b], gsb[b])

  def drain_gather(b):
    pltpu.make_async_copy(h_hbm.at[pl.ds(0, DR)], gbb[b], gsb[b]).wait()

  def apply_group(list_base, b):
    def grp(g, _):
      tv = plist[pl.ds(list_base + g * 16, 16)] & ((1 << LSH) - 1)
      for i in range(16):
        t = tv[i]
        for q in range(D_OUT // 16):
          sl = pl.ds(q * 16, 16)
          tbl[t, sl] = jnp.maximum(tbl[t, sl], gbb[b][g * 16 + i, sl])
      return 0

    lax.fori_loop(0, DR // 16, grp, 0)

  fire_dst(0, 0)

  def chunk2(ci2, cnt):
    for b in range(2):
      ci = ci2 * 2 + b

      @pl.when(ci + 1 < NCH)
      def _():
        fire_dst(ci + 1, 1 - b)

      drain_dst(b)
      dbuf = dbb[b]

      cntv0 = jnp.full((16,), cnt, jnp.int32)

      def scan(k5, cntv):
        for u in range(SCU):
          k = k5 * SCU + u
          d = dbuf[pl.ds(k * 16, 16)]
          du = d - lo
          m = du.astype(jnp.uint32) < jnp.uint32(R)
          pos = cntv + plsc.cumsum(jnp.where(m, 1, 0)) - 1
          eid = ci * CD + k * 16 + iota
          packed = lax.shift_left(eid, LSH) | du
          plsc.store_scatter(plist, [pos], packed, mask=m)
          cntv = cntv + plsc.all_reduce_population_count(m)
        return cntv

      with jax.named_scope("k4_scan"):
        cntv = lax.fori_loop(0, CD // 16 // SCU, scan, cntv0)
      cnt = cntv[0]
      nfull = cnt // DR

      @pl.when(nfull > 0)
      def _():
        fire_gather(0, 0)

      def dpair(j2, _):
        for b2 in range(2):
          j = j2 * 2 + b2

          @pl.when(j < nfull)
          def _():
            @pl.when(j + 1 < nfull)
            def _():
              fire_gather((j + 1) * DR, 1 - b2)

            drain_gather(b2)
            apply_group(j * DR, b2)
        return 0

      with jax.named_scope("k4_drain"):
        lax.fori_loop(0, (nfull + 1) // 2, dpair, 0)

      def move(v, _):
        plist[pl.ds(v * 16, 16)] = plist[pl.ds(nfull * DR + v * 16, 16)]
        return 0

      lax.fori_loop(0, DR // 16, move, 0)
      cnt = cnt - nfull * DR
    return cnt

  cnt = lax.fori_loop(0, NCH // 2, chunk2, 0)

  fire_gather(0, 0)
  drain_gather(0)

  def apply_one(i, _):
    t = plist[pl.ds(i, 16)][0] & ((1 << LSH) - 1)
    for q in range(D_OUT // 16):
      sl = pl.ds(q * 16, 16)
      tbl[t, sl] = jnp.maximum(tbl[t, sl], gb0[i, sl])
    return 0

  lax.fori_loop(0, cnt, apply_one, 0)

  pltpu.sync_copy(b2_hbm, b2v)

  def fin(i, _):
    for q in range(D_OUT // 16):
      sl = pl.ds(q * 16, 16)
      t = tbl[i, sl]
      tbl[i, sl] = jnp.where(t == -jnp.inf, 0.0, t + b2v[sl])
    return 0

  lax.fori_loop(0, R, fin, 0)
  pltpu.sync_copy(tbl, out_hbm.at[pl.ds(lo, R)])


def _sc_scatter(dst, H, b2):
  mesh = plsc.VectorSubcoreMesh(core_axis_name="c", subcore_axis_name="s")
  f = functools.partial(
      pl.kernel,
      out_type=jax.ShapeDtypeStruct((NP, D_OUT), jnp.float32),
      mesh=mesh,
      compiler_params=pltpu.CompilerParams(needs_layout_passes=False),
      scratch_types=[
          pltpu.VMEM((R, D_OUT), jnp.float32),
          pltpu.VMEM((CD,), jnp.int32),
          pltpu.VMEM((CD,), jnp.int32),
          pltpu.VMEM((LCAP,), jnp.int32),
          pltpu.VMEM((DR, D_OUT), jnp.float32),
          pltpu.VMEM((DR, D_OUT), jnp.float32),
          pltpu.VMEM((DR,), jnp.int32),
          pltpu.VMEM((DR,), jnp.int32),
          pltpu.VMEM((D_OUT,), jnp.float32),
          pltpu.SemaphoreType.DMA,
          pltpu.SemaphoreType.DMA,
          pltpu.SemaphoreType.DMA,
          pltpu.SemaphoreType.DMA,
      ],
  )(_sc_scatter_body)
  return f(dst, H, b2)


def kernel(x, W1, b1, W2, b2, edge_index):
  edge_index = edge_index.astype(jnp.int32)
  src = edge_index[0]
  dst = edge_index[1]
  P = _node_mlp(x, W1, b1)
  r = _sc_gather(dst, src, P)
  H = _mlp2(r, W2)
  out = _sc_scatter(dst, H, b2)
  return out[:N]


# speedup vs baseline: 1.2518x; 1.2518x over previous
import functools

import jax
import jax.numpy as jnp
from jax import lax
from jax.experimental import pallas as pl
from jax.experimental.pallas import tpu as pltpu
from jax.experimental.pallas import tpu_sc as plsc

N = 10000
E = 320000
D_IN = 128
D_HID = 64
D_OUT = 128

NC = 2
NS = 16
NW = NC * NS
EPW = E // NW

GC = 200
GSUB = 40
NSUB = GC // GSUB
NCHG = EPW // GC

BE = 2000

R = 320
NP = NW * R
CD = 8000
NCH = E // CD
SCU = 5
DR = 64
LCAP = 8096
LSH = 9


def _node_mlp_body(x_ref, w1_ref, b1_ref, p_ref):
  x = x_ref[...]
  w1 = w1_ref[...]
  wa = w1[:D_IN, :]
  wb = w1[D_IN:, :]
  wcat = jnp.concatenate([wa - wb, wb], axis=1)
  bcat = jnp.concatenate(
      [b1_ref[...], jnp.zeros((1, D_HID), jnp.float32)], axis=1
  )
  p_ref[...] = jnp.dot(x, wcat, preferred_element_type=jnp.float32) + bcat


def _node_mlp(x, W1, b1):
  return pl.pallas_call(
      _node_mlp_body,
      out_shape=jax.ShapeDtypeStruct((N, 2 * D_HID), jnp.float32),
  )(x, W1, b1.reshape(1, D_HID))


def _sc_gather_body(dst_hbm, src_hbm, p_hbm, r_hbm,
                    dia, sia, pd0, ps0, pd1, ps1, gs0, gs1, ws0, ws1):
  w = lax.axis_index("s") * NC + lax.axis_index("c")
  base = w * EPW
  pltpu.sync_copy(dst_hbm.at[pl.ds(base, EPW)], dia)
  pltpu.sync_copy(src_hbm.at[pl.ds(base, EPW)], sia)

  pdb = (pd0, pd1)
  psb = (ps0, ps1)
  gsb = (gs0, gs1)
  wsb = (ws0, ws1)

  def fire(ci, b):
    for k in range(NSUB):
      isl = pl.ds(ci * GC + k * GSUB, GSUB)
      sl = pl.ds(k * GSUB, GSUB)
      pltpu.async_copy(p_hbm.at[dia.at[isl]], pdb[b].at[sl], gsb[b])
      pltpu.async_copy(p_hbm.at[sia.at[isl]], psb[b].at[sl], gsb[b])

  def drain_gather(b):
    pltpu.make_async_copy(p_hbm.at[pl.ds(0, GC)], pdb[b], gsb[b]).wait()
    pltpu.make_async_copy(p_hbm.at[pl.ds(0, GC)], psb[b], gsb[b]).wait()

  def drain_wb(b):
    pltpu.make_async_copy(p_hbm.at[pl.ds(0, GC)], pdb[b], wsb[b]).wait()

  fire(0, 0)

  def step(ci2, _):
    for b in range(2):
      cur = ci2 * 2 + b

      @pl.when(cur >= 1)
      def _():
        drain_wb(1 - b)

      @pl.when(cur + 1 < NCHG)
      def _():
        fire(cur + 1, 1 - b)

      drain_gather(b)

      def relu4(i4, _):
        for r2 in range(4):
          i = i4 * 4 + r2
          for q in range(D_HID // 16):
            sl = pl.ds(q * 16, 16)
            v = pdb[b][i, sl] + psb[b][i, pl.ds(D_HID + q * 16, 16)]
            pdb[b][i, sl] = jnp.maximum(v, 0.0)
        return 0

      lax.fori_loop(0, GC // 4, relu4, 0)
      pltpu.async_copy(pdb[b], r_hbm.at[pl.ds(base + cur * GC, GC)], wsb[b])
    return 0

  lax.fori_loop(0, NCHG // 2, step, 0)
  drain_wb((NCHG - 1) % 2)


def _sc_gather(dst, src, P):
  mesh = plsc.VectorSubcoreMesh(core_axis_name="c", subcore_axis_name="s")
  f = functools.partial(
      pl.kernel,
      out_type=jax.ShapeDtypeStruct((E, 2 * D_HID), jnp.float32),
      mesh=mesh,
      compiler_params=pltpu.CompilerParams(needs_layout_passes=False),
      scratch_types=[
          pltpu.VMEM((EPW,), jnp.int32),
          pltpu.VMEM((EPW,), jnp.int32),
          pltpu.VMEM((GC, 2 * D_HID), jnp.float32),
          pltpu.VMEM((GC, 2 * D_HID), jnp.float32),
          pltpu.VMEM((GC, 2 * D_HID), jnp.float32),
          pltpu.VMEM((GC, 2 * D_HID), jnp.float32),
          pltpu.SemaphoreType.DMA,
          pltpu.SemaphoreType.DMA,
          pltpu.SemaphoreType.DMA,
          pltpu.SemaphoreType.DMA,
      ],
  )(_sc_gather_body)
  return f(dst, src, P)


def _mlp2_body(r_ref, w2_ref, h_ref):
  h_ref[...] = jnp.dot(
      r_ref[:, :D_HID], w2_ref[...], preferred_element_type=jnp.float32
  )


def _mlp2(r, W2):
  return pl.pallas_call(
      _mlp2_body,
      grid=(E // BE,),
      in_specs=[
          pl.BlockSpec((BE, 2 * D_HID), lambda i: (i, 0)),
          pl.BlockSpec((D_HID, D_OUT), lambda i: (0, 0)),
      ],
      out_specs=pl.BlockSpec((BE, D_OUT), lambda i: (i, 0)),
      out_shape=jax.ShapeDtypeStruct((E, D_OUT), jnp.float32),
  )(r, W2)


def _sc_scatter_body(dst_hbm, h_hbm, b2_hbm, out_hbm,
                     tbl, db0, db1, plist, gb0, gb1, eb0, eb1, b2v,
                     ds0, ds1, gs0, gs1):
  w = lax.axis_index("s") * NC + lax.axis_index("c")
  lo = w * R

  def init_row(i, _):
    for q in range(D_OUT // 16):
      tbl[i, pl.ds(q * 16, 16)] = jnp.full((16,), -jnp.inf, jnp.float32)
    return 0

  lax.fori_loop(0, R, init_row, 0)

  for v in range(DR // 16):
    plist[pl.ds(v * 16, 16)] = jnp.zeros((16,), jnp.int32)

  dbb = (db0, db1)
  dsb = (ds0, ds1)
  gbb = (gb0, gb1)
  gsb = (gs0, gs1)
  iota = lax.iota(jnp.int32, 16)

  def fire_dst(ci, b):
    pltpu.async_copy(dst_hbm.at[pl.ds(ci * CD, CD)], dbb[b], dsb[b])

  def drain_dst(b):
    pltpu.make_async_copy(dst_hbm.at[pl.ds(0, CD)], dbb[b], dsb[b]).wait()

  ebb = (eb0, eb1)

  def fire_gather(list_base, b):
    for v in range(DR // 16):
      pv = plist[pl.ds(list_base + v * 16, 16)]
      ebb[b][pl.ds(v * 16, 16)] = lax.shift_right_logical(pv, LSH)
    pltpu.async_copy(h_hbm.at[ebb[b]], gbb[b], gsb[b])

  def drain_gather(b):
    pltpu.make_async_copy(h_hbm.at[pl.ds(0, DR)], gbb[b], gsb[b]).wait()

  def apply_group(list_base, b):
    def grp(g, _):
      tv = plist[pl.ds(list_base + g * 16, 16)] & ((1 << LSH) - 1)
      for i in range(16):
        t = tv[i]
        tl = [tbl[t, pl.ds(q * 16, 16)] for q in range(D_OUT // 16)]
        gl = [gbb[b][g * 16 + i, pl.ds(q * 16, 16)]
              for q in range(D_OUT // 16)]
        for q in range(D_OUT // 16):
          tbl[t, pl.ds(q * 16, 16)] = jnp.maximum(tl[q], gl[q])
      return 0

    lax.fori_loop(0, DR // 16, grp, 0)

  fire_dst(0, 0)

  def chunk2(ci2, cnt):
    for b in range(2):
      ci = ci2 * 2 + b

      @pl.when(ci + 1 < NCH)
      def _():
        fire_dst(ci + 1, 1 - b)

      drain_dst(b)
      dbuf = dbb[b]

      cntm0 = jnp.full((16,), cnt - 1, jnp.int32)
      bsh0 = lax.shift_left(ci * CD + iota, LSH)

      def scan(k5, carry):
        cntm, bsh = carry
        for u in range(SCU):
          d = dbuf[pl.ds((k5 * SCU + u) * 16, 16)]
          du = d - lo
          m = du.astype(jnp.uint32) < jnp.uint32(R)
          pos = cntm + plsc.cumsum(jnp.where(m, 1, 0))
          plsc.store_scatter(plist, [pos], bsh + du, mask=m)
          cntm = cntm + plsc.all_reduce_population_count(m)
          bsh = bsh + (16 << LSH)
        return cntm, bsh

      with jax.named_scope("k4_scan"):
        cntm, _ = lax.fori_loop(0, CD // 16 // SCU, scan, (cntm0, bsh0))
      cnt = cntm[0] + 1
      nfull = cnt // DR

      @pl.when(nfull > 0)
      def _():
        fire_gather(0, 0)

      def dpair(j2, _):
        for b2 in range(2):
          j = j2 * 2 + b2

          @pl.when(j < nfull)
          def _():
            @pl.when(j + 1 < nfull)
            def _():
              fire_gather((j + 1) * DR, 1 - b2)

            drain_gather(b2)
            apply_group(j * DR, b2)
        return 0

      with jax.named_scope("k4_drain"):
        lax.fori_loop(0, (nfull + 1) // 2, dpair, 0)

      def move(v, _):
        plist[pl.ds(v * 16, 16)] = plist[pl.ds(nfull * DR + v * 16, 16)]
        return 0

      lax.fori_loop(0, DR // 16, move, 0)
      cnt = cnt - nfull * DR
    return cnt

  cnt = lax.fori_loop(0, NCH // 2, chunk2, 0)

  fire_gather(0, 0)
  drain_gather(0)

  def apply_one(i, _):
    t = plist[pl.ds(i, 16)][0] & ((1 << LSH) - 1)
    tl = [tbl[t, pl.ds(q * 16, 16)] for q in range(D_OUT // 16)]
    gl = [gb0[i, pl.ds(q * 16, 16)] for q in range(D_OUT // 16)]
    for q in range(D_OUT // 16):
      tbl[t, pl.ds(q * 16, 16)] = jnp.maximum(tl[q], gl[q])
    return 0

  lax.fori_loop(0, cnt, apply_one, 0)

  pltpu.sync_copy(b2_hbm, b2v)

  def fin(i, _):
    for q in range(D_OUT // 16):
      sl = pl.ds(q * 16, 16)
      t = tbl[i, sl]
      tbl[i, sl] = jnp.where(t == -jnp.inf, 0.0, t + b2v[sl])
    return 0

  lax.fori_loop(0, R, fin, 0)
  pltpu.sync_copy(tbl, out_hbm.at[pl.ds(lo, R)])


def _sc_scatter(dst, H, b2):
  mesh = plsc.VectorSubcoreMesh(core_axis_name="c", subcore_axis_name="s")
  f = functools.partial(
      pl.kernel,
      out_type=jax.ShapeDtypeStruct((NP, D_OUT), jnp.float32),
      mesh=mesh,
      compiler_params=pltpu.CompilerParams(needs_layout_passes=False),
      scratch_types=[
          pltpu.VMEM((R, D_OUT), jnp.float32),
          pltpu.VMEM((CD,), jnp.int32),
          pltpu.VMEM((CD,), jnp.int32),
          pltpu.VMEM((LCAP,), jnp.int32),
          pltpu.VMEM((DR, D_OUT), jnp.float32),
          pltpu.VMEM((DR, D_OUT), jnp.float32),
          pltpu.VMEM((DR,), jnp.int32),
          pltpu.VMEM((DR,), jnp.int32),
          pltpu.VMEM((D_OUT,), jnp.float32),
          pltpu.SemaphoreType.DMA,
          pltpu.SemaphoreType.DMA,
          pltpu.SemaphoreType.DMA,
          pltpu.SemaphoreType.DMA,
      ],
  )(_sc_scatter_body)
  return f(dst, H, b2)


def kernel(x, W1, b1, W2, b2, edge_index):
  edge_index = edge_index.astype(jnp.int32)
  src = edge_index[0]
  dst = edge_index[1]
  P = _node_mlp(x, W1, b1)
  r = _sc_gather(dst, src, P)
  H = _mlp2(r, W2)
  out = _sc_scatter(dst, H, b2)
  return out[:N]


# speedup vs baseline: 1.6185x; 1.2929x over previous
import functools

import jax
import jax.numpy as jnp
from jax import lax
from jax.experimental import pallas as pl
from jax.experimental.pallas import tpu as pltpu
from jax.experimental.pallas import tpu_sc as plsc

N = 10000
E = 320000
D_IN = 128
D_HID = 64
D_OUT = 128

NC = 2
NS = 16
NW = NC * NS
EPW = E // NW

GC = 200
GSUB = 40
NSUB = GC // GSUB
NCHG = EPW // GC

BE = 2000

R = 320
NP = NW * R
CD = 8000
NCH = E // CD
SCU = 10
DR = 64
LCAP = 8096
LSH = 9


def _node_mlp_body(x_ref, w1_ref, b1_ref, p_ref):
  x = x_ref[...]
  w1 = w1_ref[...]
  wa = w1[:D_IN, :]
  wb = w1[D_IN:, :]
  wcat = jnp.concatenate([wa - wb, wb], axis=1)
  bcat = jnp.concatenate(
      [b1_ref[...], jnp.zeros((1, D_HID), jnp.float32)], axis=1
  )
  p_ref[...] = jnp.dot(x, wcat, preferred_element_type=jnp.float32) + bcat


def _node_mlp(x, W1, b1):
  return pl.pallas_call(
      _node_mlp_body,
      out_shape=jax.ShapeDtypeStruct((N, 2 * D_HID), jnp.float32),
  )(x, W1, b1.reshape(1, D_HID))


def _sc_gather_body(dst_hbm, src_hbm, p_hbm, r_hbm,
                    dia, sia, pd0, ps0, pd1, ps1, gs0, gs1, ws0, ws1):
  w = lax.axis_index("s") * NC + lax.axis_index("c")
  base = w * EPW
  pltpu.sync_copy(dst_hbm.at[pl.ds(base, EPW)], dia)
  pltpu.sync_copy(src_hbm.at[pl.ds(base, EPW)], sia)

  pdb = (pd0, pd1)
  psb = (ps0, ps1)
  gsb = (gs0, gs1)
  wsb = (ws0, ws1)

  def fire(ci, b):
    for k in range(NSUB):
      isl = pl.ds(ci * GC + k * GSUB, GSUB)
      sl = pl.ds(k * GSUB, GSUB)
      pltpu.async_copy(p_hbm.at[dia.at[isl]], pdb[b].at[sl], gsb[b])
      pltpu.async_copy(p_hbm.at[sia.at[isl]], psb[b].at[sl], gsb[b])

  def drain_gather(b):
    pltpu.make_async_copy(p_hbm.at[pl.ds(0, GC)], pdb[b], gsb[b]).wait()
    pltpu.make_async_copy(p_hbm.at[pl.ds(0, GC)], psb[b], gsb[b]).wait()

  def drain_wb(b):
    pltpu.make_async_copy(p_hbm.at[pl.ds(0, GC)], pdb[b], wsb[b]).wait()

  fire(0, 0)

  def step(ci2, _):
    for b in range(2):
      cur = ci2 * 2 + b

      @pl.when(cur >= 1)
      def _():
        drain_wb(1 - b)

      @pl.when(cur + 1 < NCHG)
      def _():
        fire(cur + 1, 1 - b)

      drain_gather(b)

      def relu4(i4, _):
        for r2 in range(4):
          i = i4 * 4 + r2
          for q in range(D_HID // 16):
            sl = pl.ds(q * 16, 16)
            v = pdb[b][i, sl] + psb[b][i, pl.ds(D_HID + q * 16, 16)]
            pdb[b][i, sl] = jnp.maximum(v, 0.0)
        return 0

      lax.fori_loop(0, GC // 4, relu4, 0)
      pltpu.async_copy(pdb[b], r_hbm.at[pl.ds(base + cur * GC, GC)], wsb[b])
    return 0

  lax.fori_loop(0, NCHG // 2, step, 0)
  drain_wb((NCHG - 1) % 2)


def _sc_gather(dst, src, P):
  mesh = plsc.VectorSubcoreMesh(core_axis_name="c", subcore_axis_name="s")
  f = functools.partial(
      pl.kernel,
      out_type=jax.ShapeDtypeStruct((E, 2 * D_HID), jnp.float32),
      mesh=mesh,
      compiler_params=pltpu.CompilerParams(needs_layout_passes=False),
      scratch_types=[
          pltpu.VMEM((EPW,), jnp.int32),
          pltpu.VMEM((EPW,), jnp.int32),
          pltpu.VMEM((GC, 2 * D_HID), jnp.float32),
          pltpu.VMEM((GC, 2 * D_HID), jnp.float32),
          pltpu.VMEM((GC, 2 * D_HID), jnp.float32),
          pltpu.VMEM((GC, 2 * D_HID), jnp.float32),
          pltpu.SemaphoreType.DMA,
          pltpu.SemaphoreType.DMA,
          pltpu.SemaphoreType.DMA,
          pltpu.SemaphoreType.DMA,
      ],
  )(_sc_gather_body)
  return f(dst, src, P)


def _mlp2_body(r_ref, w2_ref, h_ref):
  h_ref[...] = jnp.dot(
      r_ref[:, :D_HID], w2_ref[...], preferred_element_type=jnp.float32
  )


def _mlp2(r, W2):
  return pl.pallas_call(
      _mlp2_body,
      grid=(E // BE,),
      in_specs=[
          pl.BlockSpec((BE, 2 * D_HID), lambda i: (i, 0)),
          pl.BlockSpec((D_HID, D_OUT), lambda i: (0, 0)),
      ],
      out_specs=pl.BlockSpec((BE, D_OUT), lambda i: (i, 0)),
      out_shape=jax.ShapeDtypeStruct((E, D_OUT), jnp.float32),
  )(r, W2)


def _sc_scatter_body(dst_hbm, h_hbm, b2_hbm, out_hbm,
                     tbl, db0, db1, plist, gb0, gb1, eb0, eb1, b2v,
                     ds0, ds1, gs0, gs1):
  w = lax.axis_index("s") * NC + lax.axis_index("c")
  lo = w * R

  def init_row(i, _):
    for q in range(D_OUT // 16):
      tbl[i, pl.ds(q * 16, 16)] = jnp.full((16,), -jnp.inf, jnp.float32)
    return 0

  lax.fori_loop(0, R, init_row, 0)

  for v in range(DR // 16):
    plist[pl.ds(v * 16, 16)] = jnp.zeros((16,), jnp.int32)

  dbb = (db0, db1)
  dsb = (ds0, ds1)
  gbb = (gb0, gb1)
  gsb = (gs0, gs1)
  iota = lax.iota(jnp.int32, 16)

  def fire_dst(ci, b):
    pltpu.async_copy(dst_hbm.at[pl.ds(ci * CD, CD)], dbb[b], dsb[b])

  def drain_dst(b):
    pltpu.make_async_copy(dst_hbm.at[pl.ds(0, CD)], dbb[b], dsb[b]).wait()

  ebb = (eb0, eb1)

  def fire_gather(list_base, b):
    for v in range(DR // 16):
      pv = plist[pl.ds(list_base + v * 16, 16)]
      ebb[b][pl.ds(v * 16, 16)] = lax.shift_right_logical(pv, LSH)
    pltpu.async_copy(h_hbm.at[ebb[b]], gbb[b], gsb[b])

  def drain_gather(b):
    pltpu.make_async_copy(h_hbm.at[pl.ds(0, DR)], gbb[b], gsb[b]).wait()

  def apply_group(list_base, b):
    def grp(g, _):
      tv = plist[pl.ds(list_base + g * 16, 16)] & ((1 << LSH) - 1)
      for i in range(16):
        t = tv[i]
        tl = [tbl[t, pl.ds(q * 16, 16)] for q in range(D_OUT // 16)]
        gl = [gbb[b][g * 16 + i, pl.ds(q * 16, 16)]
              for q in range(D_OUT // 16)]
        for q in range(D_OUT // 16):
          tbl[t, pl.ds(q * 16, 16)] = jnp.maximum(tl[q], gl[q])
      return 0

    lax.fori_loop(0, DR // 16, grp, 0)

  fire_dst(0, 0)

  def chunk2(ci2, cnt):
    for b in range(2):
      ci = ci2 * 2 + b

      @pl.when(ci + 1 < NCH)
      def _():
        fire_dst(ci + 1, 1 - b)

      drain_dst(b)
      dbuf = dbb[b]

      cntm0 = jnp.full((16,), cnt - 1, jnp.int32)
      bsh0 = lax.shift_left(ci * CD + iota, LSH)

      def scan(k5, carry):
        cntm, bsh = carry
        dus, ms, css, pcs = [], [], [], []
        for u in range(SCU):
          d = dbuf[pl.ds((k5 * SCU + u) * 16, 16)]
          du = d - lo
          m = du.astype(jnp.uint32) < jnp.uint32(R)
          dus.append(du)
          ms.append(m)
          css.append(plsc.cumsum(jnp.where(m, 1, 0)))
          pcs.append(plsc.all_reduce_population_count(m))
        for u in range(SCU):
          pos = cntm + css[u]
          plsc.store_scatter(plist, [pos], bsh + (dus[u] + ((u * 16) << LSH)),
                             mask=ms[u])
          cntm = cntm + pcs[u]
        return cntm, bsh + ((SCU * 16) << LSH)

      with jax.named_scope("k4_scan"):
        cntm, _ = lax.fori_loop(0, CD // 16 // SCU, scan, (cntm0, bsh0))
      cnt = cntm[0] + 1
      nfull = cnt // DR

      @pl.when(nfull > 0)
      def _():
        fire_gather(0, 0)

      def dpair(j2, _):
        for b2 in range(2):
          j = j2 * 2 + b2

          @pl.when(j < nfull)
          def _():
            @pl.when(j + 1 < nfull)
            def _():
              fire_gather((j + 1) * DR, 1 - b2)

            drain_gather(b2)
            apply_group(j * DR, b2)
        return 0

      with jax.named_scope("k4_drain"):
        lax.fori_loop(0, (nfull + 1) // 2, dpair, 0)

      def move(v, _):
        plist[pl.ds(v * 16, 16)] = plist[pl.ds(nfull * DR + v * 16, 16)]
        return 0

      lax.fori_loop(0, DR // 16, move, 0)
      cnt = cnt - nfull * DR
    return cnt

  cnt = lax.fori_loop(0, NCH // 2, chunk2, 0)

  fire_gather(0, 0)
  drain_gather(0)

  def apply_one(i, _):
    t = plist[pl.ds(i, 16)][0] & ((1 << LSH) - 1)
    tl = [tbl[t, pl.ds(q * 16, 16)] for q in range(D_OUT // 16)]
    gl = [gb0[i, pl.ds(q * 16, 16)] for q in range(D_OUT // 16)]
    for q in range(D_OUT // 16):
      tbl[t, pl.ds(q * 16, 16)] = jnp.maximum(tl[q], gl[q])
    return 0

  lax.fori_loop(0, cnt, apply_one, 0)

  pltpu.sync_copy(b2_hbm, b2v)

  def fin(i, _):
    for q in range(D_OUT // 16):
      sl = pl.ds(q * 16, 16)
      t = tbl[i, sl]
      tbl[i, sl] = jnp.where(t == -jnp.inf, 0.0, t + b2v[sl])
    return 0

  lax.fori_loop(0, R, fin, 0)
  pltpu.sync_copy(tbl, out_hbm.at[pl.ds(lo, R)])


def _sc_scatter(dst, H, b2):
  mesh = plsc.VectorSubcoreMesh(core_axis_name="c", subcore_axis_name="s")
  f = functools.partial(
      pl.kernel,
      out_type=jax.ShapeDtypeStruct((NP, D_OUT), jnp.float32),
      mesh=mesh,
      compiler_params=pltpu.CompilerParams(needs_layout_passes=False),
      scratch_types=[
          pltpu.VMEM((R, D_OUT), jnp.float32),
          pltpu.VMEM((CD,), jnp.int32),
          pltpu.VMEM((CD,), jnp.int32),
          pltpu.VMEM((LCAP,), jnp.int32),
          pltpu.VMEM((DR, D_OUT), jnp.float32),
          pltpu.VMEM((DR, D_OUT), jnp.float32),
          pltpu.VMEM((DR,), jnp.int32),
          pltpu.VMEM((DR,), jnp.int32),
          pltpu.VMEM((D_OUT,), jnp.float32),
          pltpu.SemaphoreType.DMA,
          pltpu.SemaphoreType.DMA,
          pltpu.SemaphoreType.DMA,
          pltpu.SemaphoreType.DMA,
      ],
  )(_sc_scatter_body)
  return f(dst, H, b2)


def kernel(x, W1, b1, W2, b2, edge_index):
  edge_index = edge_index.astype(jnp.int32)
  src = edge_index[0]
  dst = edge_index[1]
  P = _node_mlp(x, W1, b1)
  r = _sc_gather(dst, src, P)
  H = _mlp2(r, W2)
  out = _sc_scatter(dst, H, b2)
  return out[:N]
